# Initial kernel scaffold; baseline (speedup 1.0000x reference)
import jax
import jax.numpy as jnp
from jax.experimental import pallas as pl


def kernel(proposals, logits):
    raise NotImplementedError("write your pallas kernel here")


# speedup vs baseline: 19.4377x; 19.4377x over previous
import functools

import jax
import jax.numpy as jnp
from jax.experimental import pallas as pl
from jax.experimental.pallas import tpu as pltpu

_PRE_TOPK = 2000
_POST_TOPK = 1000
_N = 2048
_KOUT = 1024
_THRESH = 0.7
_IMG_H = 1024.0
_IMG_W = 1024.0
_CHUNK = 2048


def _nms_body(props_ref, tk_ref, out_ref, mu_ref):
    props = props_ref[0]
    s = tk_ref[0, 0:1, :]
    idxf = tk_ref[0, 1:2, :]

    a_pad = props.shape[1]
    n_chunks = a_pad // _CHUNK

    g = jnp.zeros((8, _N), jnp.float32)
    for c in range(n_chunks):
        base = c * _CHUNK
        a_col = jax.lax.broadcasted_iota(jnp.int32, (_CHUNK, _N), 0) + base
        oh = jnp.where(a_col.astype(jnp.float32) == idxf, 1.0, 0.0)
        g = g + jax.lax.dot_general(
            props[:, base:base + _CHUNK], oh,
            (((1,), (0,)), ((), ())),
            precision=jax.lax.Precision.HIGHEST,
            preferred_element_type=jnp.float32)

    cx = jnp.clip(g[0:1, :], 0.0, _IMG_W)
    cy = jnp.clip(g[1:2, :], 0.0, _IMG_H)
    w = g[2:3, :]
    h = g[3:4, :]
    ang = g[4:5, :]

    x1 = cx - w * 0.5
    y1 = cy - h * 0.5
    x2 = cx + w * 0.5
    y2 = cy + h * 0.5
    area = (x2 - x1) * (y2 - y1)

    fin = (jnp.isfinite(cx) & jnp.isfinite(cy) & jnp.isfinite(w)
           & jnp.isfinite(h) & jnp.isfinite(ang) & jnp.isfinite(s))
    valid = (w > 0.0) & (h > 0.0) & fin
    s_eff = jnp.where(valid, s, -1e9)

    colT = jnp.transpose(
        jnp.concatenate([x1, y1, x2, y2, area,
                         jnp.zeros((3, _N), jnp.float32)], axis=0))

    rb_size = 256
    for rb in range(_N // rb_size):
        lo, hi = rb * rb_size, (rb + 1) * rb_size
        x1j = colT[lo:hi, 0:1]
        y1j = colT[lo:hi, 1:2]
        x2j = colT[lo:hi, 2:3]
        y2j = colT[lo:hi, 3:4]
        aj = colT[lo:hi, 4:5]
        ix1 = jnp.maximum(x1j, x1)
        iy1 = jnp.maximum(y1j, y1)
        ix2 = jnp.minimum(x2j, x2)
        iy2 = jnp.minimum(y2j, y2)
        iw = jnp.clip(ix2 - ix1, 0.0)
        ih = jnp.clip(iy2 - iy1, 0.0)
        inter = iw * ih
        union = aj + area - inter
        iou = inter / (union + 1e-9)
        jg = jax.lax.broadcasted_iota(jnp.int32, (rb_size, _N), 0) + rb * rb_size
        ii = jax.lax.broadcasted_iota(jnp.int32, (rb_size, _N), 1)
        m = (iou > _THRESH) & (ii > jg)
        mu_ref[lo:hi, :] = jnp.where(m, 1.0, 0.0)

    def _cond(carry):
        return carry[1]

    def _body(carry):
        keep, _ = carry
        sup = jax.lax.dot_general(
            keep, mu_ref[:, :], (((1,), (0,)), ((), ())),
            preferred_element_type=jnp.float32)
        newkeep = jnp.where(sup > 0.5, 0.0, 1.0)
        changed = jnp.any(newkeep != keep)
        return newkeep, changed

    keep0 = jnp.ones((1, _N), jnp.float32)
    keep, _ = jax.lax.while_loop(_cond, _body, (keep0, True))

    real = jax.lax.broadcasted_iota(jnp.int32, (1, _N), 1) < _PRE_TOPK
    kept = jnp.where(real, keep, 0.0)
    nk = jnp.sum(kept)

    for rb in range(_N // rb_size):
        lo, hi = rb * rb_size, (rb + 1) * rb_size
        jg = jax.lax.broadcasted_iota(jnp.int32, (rb_size, _N), 0) + lo
        ii = jax.lax.broadcasted_iota(jnp.int32, (rb_size, _N), 1)
        mu_ref[lo:hi, :] = jnp.where(jg <= ii, 1.0, 0.0)
    csk = jax.lax.dot_general(
        kept, mu_ref[:, :], (((1,), (0,)), ((), ())),
        preferred_element_type=jnp.float32)
    idx_row = jax.lax.broadcasted_iota(
        jnp.int32, (1, _N), 1).astype(jnp.float32)
    csn = (idx_row + 1.0) - csk
    pos = jnp.where(kept > 0.5, csk - 1.0, nk + csn - 1.0)

    p_col = jax.lax.broadcasted_iota(jnp.int32, (_KOUT, _N), 0).astype(jnp.float32)
    perm = jnp.where(p_col == pos, 1.0, 0.0)

    data = jnp.concatenate(
        [cx, cy, w, h, ang, s_eff, jnp.zeros((2, _N), jnp.float32)], axis=0)
    out = jax.lax.dot_general(
        data, perm, (((1,), (1,)), ((), ())),
        precision=jax.lax.Precision.HIGHEST,
        preferred_element_type=jnp.float32)
    out_ref[0] = out


@jax.jit
def kernel(proposals, logits):
    B, A = logits.shape
    scores, idx = jax.lax.top_k(logits, _PRE_TOPK)

    pad_n = _N - _PRE_TOPK
    s_p = jnp.pad(scores, ((0, 0), (0, pad_n)), constant_values=-1e9)
    i_p = jnp.pad(idx.astype(jnp.float32), ((0, 0), (0, pad_n)),
                  constant_values=-1.0)
    tk = jnp.zeros((B, 8, _N), jnp.float32)
    tk = tk.at[:, 0, :].set(s_p)
    tk = tk.at[:, 1, :].set(i_p)

    a_pad = ((A + _CHUNK - 1) // _CHUNK) * _CHUNK
    props_t = jnp.transpose(proposals, (0, 2, 1))
    props_t = jnp.pad(props_t, ((0, 0), (0, 3), (0, a_pad - A)))

    out_t = pl.pallas_call(
        _nms_body,
        grid=(B,),
        in_specs=[
            pl.BlockSpec((1, 8, a_pad), lambda b: (b, 0, 0)),
            pl.BlockSpec((1, 8, _N), lambda b: (b, 0, 0)),
        ],
        out_specs=pl.BlockSpec((1, 8, _KOUT), lambda b: (b, 0, 0)),
        out_shape=jax.ShapeDtypeStruct((B, 8, _KOUT), jnp.float32),
        scratch_shapes=[pltpu.VMEM((_N, _N), jnp.float32)],
    )(props_t, tk)

    return jnp.transpose(out_t, (0, 2, 1))[:, :_POST_TOPK, :6]


# speedup vs baseline: 19.7782x; 1.0175x over previous
import functools

import jax
import jax.numpy as jnp
from jax.experimental import pallas as pl
from jax.experimental.pallas import tpu as pltpu

_PRE_TOPK = 2000
_POST_TOPK = 1000
_N = 2048
_KOUT = 1024
_THRESH = 0.7
_IMG_H = 1024.0
_IMG_W = 1024.0
_CHUNK = 2048


def _nms_body(props_ref, tk_ref, out_ref, mu_ref):
    props = props_ref[0]
    s = tk_ref[0, 0:1, :]
    idxf = tk_ref[0, 1:2, :]

    a_pad = props.shape[1]
    n_chunks = a_pad // _CHUNK

    g = jnp.zeros((8, _N), jnp.float32)
    a_col = jax.lax.broadcasted_iota(
        jnp.int32, (_CHUNK, _N), 0).astype(jnp.float32)
    for c in range(n_chunks):
        base = c * _CHUNK
        oh = jnp.where(a_col == idxf - float(base), 1.0, 0.0)
        g = g + jax.lax.dot_general(
            props[:, base:base + _CHUNK], oh,
            (((1,), (0,)), ((), ())),
            precision=jax.lax.Precision.HIGHEST,
            preferred_element_type=jnp.float32)

    cx = jnp.clip(g[0:1, :], 0.0, _IMG_W)
    cy = jnp.clip(g[1:2, :], 0.0, _IMG_H)
    w = g[2:3, :]
    h = g[3:4, :]
    ang = g[4:5, :]

    x1 = cx - w * 0.5
    y1 = cy - h * 0.5
    x2 = cx + w * 0.5
    y2 = cy + h * 0.5
    area = (x2 - x1) * (y2 - y1)

    fin = (jnp.isfinite(cx) & jnp.isfinite(cy) & jnp.isfinite(w)
           & jnp.isfinite(h) & jnp.isfinite(ang) & jnp.isfinite(s))
    valid = (w > 0.0) & (h > 0.0) & fin
    s_eff = jnp.where(valid, s, -1e9)

    colT = jnp.transpose(
        jnp.concatenate([x1, y1, x2, y2, area,
                         jnp.zeros((3, _N), jnp.float32)], axis=0))

    rb_size = 256
    for rb in range(_N // rb_size):
        lo, hi = rb * rb_size, (rb + 1) * rb_size
        x1j = colT[lo:hi, 0:1]
        y1j = colT[lo:hi, 1:2]
        x2j = colT[lo:hi, 2:3]
        y2j = colT[lo:hi, 3:4]
        aj = colT[lo:hi, 4:5]
        ix1 = jnp.maximum(x1j, x1)
        iy1 = jnp.maximum(y1j, y1)
        ix2 = jnp.minimum(x2j, x2)
        iy2 = jnp.minimum(y2j, y2)
        iw = jnp.clip(ix2 - ix1, 0.0)
        ih = jnp.clip(iy2 - iy1, 0.0)
        inter = iw * ih
        union = aj + area - inter
        iou = inter / (union + 1e-9)
        jg = jax.lax.broadcasted_iota(jnp.int32, (rb_size, _N), 0) + rb * rb_size
        ii = jax.lax.broadcasted_iota(jnp.int32, (rb_size, _N), 1)
        m = (iou > _THRESH) & (ii > jg)
        mu_ref[lo:hi, :] = jnp.where(m, 1.0, 0.0).astype(jnp.bfloat16)

    def _cond(carry):
        return carry[1]

    def _body(carry):
        keep, _ = carry
        sup = jax.lax.dot_general(
            keep, mu_ref[:, :], (((1,), (0,)), ((), ())),
            preferred_element_type=jnp.float32)
        newkeep = jnp.where(sup > 0.5, 0.0, 1.0).astype(jnp.bfloat16)
        d = newkeep.astype(jnp.float32) - keep.astype(jnp.float32)
        changed = jnp.sum(d * d) > 0.0
        return newkeep, changed

    keep0 = jnp.ones((1, _N), jnp.bfloat16)
    keep, _ = jax.lax.while_loop(_cond, _body, (keep0, True))
    keep = keep.astype(jnp.float32)

    real = jax.lax.broadcasted_iota(jnp.int32, (1, _N), 1) < _PRE_TOPK
    kept = jnp.where(real, keep, 0.0)
    nk = jnp.sum(kept)

    for rb in range(_N // rb_size):
        lo, hi = rb * rb_size, (rb + 1) * rb_size
        jg = jax.lax.broadcasted_iota(jnp.int32, (rb_size, _N), 0) + lo
        ii = jax.lax.broadcasted_iota(jnp.int32, (rb_size, _N), 1)
        mu_ref[lo:hi, :] = jnp.where(jg <= ii, 1.0, 0.0).astype(jnp.bfloat16)
    csk = jax.lax.dot_general(
        kept.astype(jnp.bfloat16), mu_ref[:, :], (((1,), (0,)), ((), ())),
        preferred_element_type=jnp.float32)
    idx_row = jax.lax.broadcasted_iota(
        jnp.int32, (1, _N), 1).astype(jnp.float32)
    csn = (idx_row + 1.0) - csk
    pos = jnp.where(kept > 0.5, csk - 1.0, nk + csn - 1.0)

    p_col = jax.lax.broadcasted_iota(jnp.int32, (_KOUT, _N), 0).astype(jnp.float32)
    perm = jnp.where(p_col == pos, 1.0, 0.0)

    data = jnp.concatenate(
        [cx, cy, w, h, ang, s_eff, jnp.zeros((2, _N), jnp.float32)], axis=0)
    out = jax.lax.dot_general(
        data, perm, (((1,), (1,)), ((), ())),
        precision=jax.lax.Precision.HIGHEST,
        preferred_element_type=jnp.float32)
    out_ref[0] = out


@jax.jit
def kernel(proposals, logits):
    B, A = logits.shape
    scores, idx = jax.lax.top_k(logits, _PRE_TOPK)

    pad_n = _N - _PRE_TOPK
    s_p = jnp.pad(scores, ((0, 0), (0, pad_n)), constant_values=-1e9)
    i_p = jnp.pad(idx.astype(jnp.float32), ((0, 0), (0, pad_n)),
                  constant_values=-1.0)
    tk = jnp.zeros((B, 8, _N), jnp.float32)
    tk = tk.at[:, 0, :].set(s_p)
    tk = tk.at[:, 1, :].set(i_p)

    a_pad = ((A + _CHUNK - 1) // _CHUNK) * _CHUNK
    props_t = jnp.transpose(proposals, (0, 2, 1))
    props_t = jnp.pad(props_t, ((0, 0), (0, 3), (0, a_pad - A)))

    out_t = pl.pallas_call(
        _nms_body,
        grid=(B,),
        in_specs=[
            pl.BlockSpec((1, 8, a_pad), lambda b: (b, 0, 0)),
            pl.BlockSpec((1, 8, _N), lambda b: (b, 0, 0)),
        ],
        out_specs=pl.BlockSpec((1, 8, _KOUT), lambda b: (b, 0, 0)),
        out_shape=jax.ShapeDtypeStruct((B, 8, _KOUT), jnp.float32),
        scratch_shapes=[pltpu.VMEM((_N, _N), jnp.bfloat16)],
    )(props_t, tk)

    return jnp.transpose(out_t, (0, 2, 1))[:, :_POST_TOPK, :6]
